# Initial kernel scaffold; baseline (speedup 1.0000x reference)
import jax
import jax.numpy as jnp
from jax import lax
from jax.experimental import pallas as pl
from jax.experimental.pallas import tpu as pltpu

_B, _C, _H, _W = 8, 384, 14, 14
_E, _K, _R = 8, 2, 4
_T = _B * _H * _W
_HID = _R * _C
_CAP = int(1.25 * _T * _K / _E)
_TB = 392
_RB = 112


def _stage1_body(xh_ref, dwk_ref, dwb_ref, lnw_ref, lnb_ref, rw_ref,
                 xn_ref, gates_ref):
    x = xh_ref[...]
    zh = jnp.zeros((_B, 3, _W, _C), jnp.float32)
    xp = jnp.concatenate([zh, x, zh], axis=1)
    zw = jnp.zeros((_B, _H + 6, 3, _C), jnp.float32)
    xp = jnp.concatenate([zw, xp, zw], axis=2)
    acc = jnp.zeros((_B, _H, _W, _C), jnp.float32)
    for i in range(7):
        for j in range(7):
            acc = acc + xp[:, i:i + _H, j:j + _W, :] * dwk_ref[i, j, :]
    y = (acc + dwb_ref[...]).reshape(_T, _C)

    mu = jnp.mean(y, axis=1, keepdims=True)
    var = jnp.mean((y - mu) * (y - mu), axis=1, keepdims=True)
    xn = (y - mu) * lax.rsqrt(var + 1e-6) * lnw_ref[...] + lnb_ref[...]
    xn_ref[...] = xn

    logits = jnp.dot(xn, rw_ref[...], preferred_element_type=jnp.float32)
    m = jnp.max(logits, axis=1, keepdims=True)
    ex = jnp.exp(logits - m)
    probs = ex / jnp.sum(ex, axis=1, keepdims=True)
    e_iota = lax.broadcasted_iota(jnp.int32, (_T, _E), 1)
    p1 = jnp.max(probs, axis=1, keepdims=True)
    i1 = jnp.min(jnp.where(probs == p1, e_iota, _E), axis=1, keepdims=True)
    probs2 = jnp.where(e_iota == i1, -1.0, probs)
    p2 = jnp.max(probs2, axis=1, keepdims=True)
    i2 = jnp.min(jnp.where(probs2 == p2, e_iota, _E), axis=1, keepdims=True)
    denom = p1 + p2
    w1v = p1 / denom
    w2v = p2 / denom

    maskE = ((e_iota == i1) | (e_iota == i2)).astype(jnp.float32)
    prio_row = p1.reshape(1, _T)
    rank_rows = []
    for bi in range(_T // _RB):
        pr = p1[bi * _RB:(bi + 1) * _RB, :]
        rows_t = lax.broadcasted_iota(jnp.int32, (_RB, _T), 0) + bi * _RB
        cols_t = lax.broadcasted_iota(jnp.int32, (_RB, _T), 1)
        cmpb = ((prio_row > pr) | ((prio_row == pr) & (cols_t < rows_t)))
        rankb = jnp.dot(cmpb.astype(jnp.float32), maskE,
                        preferred_element_type=jnp.float32)
        rank_rows.append(rankb)
    rank = jnp.concatenate(rank_rows, axis=0)

    r1 = jnp.sum(rank * (e_iota == i1), axis=1, keepdims=True)
    r2 = jnp.sum(rank * (e_iota == i2), axis=1, keepdims=True)
    g1 = jnp.where(r1 < _CAP, w1v, 0.0)
    g2 = jnp.where(r2 < _CAP, w2v, 0.0)
    gates_ref[...] = g1 * (e_iota == i1) + g2 * (e_iota == i2)


def _expert_body(xn_ref, w1_ref, b1_ref, w2_ref, b2_ref, gates_ref,
                 xh_ref, ls_ref, out_ref):
    e = pl.program_id(1)
    h = jnp.dot(xn_ref[...], w1_ref[0], preferred_element_type=jnp.float32)
    h = h + b1_ref[...]
    h = 0.5 * h * (1.0 + lax.erf(h * 0.7071067811865476))
    y = jnp.dot(h, w2_ref[0], preferred_element_type=jnp.float32) + b2_ref[...]
    sel = (lax.broadcasted_iota(jnp.int32, (_TB, _E), 1) == e)
    ge = jnp.sum(jnp.where(sel, gates_ref[...], 0.0), axis=1, keepdims=True)
    contrib = ge * y

    @pl.when(e == 0)
    def _():
        out_ref[...] = contrib

    @pl.when(e != 0)
    def _():
        out_ref[...] += contrib

    @pl.when(e == _E - 1)
    def _():
        out_ref[...] = xh_ref[...] + xn_ref[...] + ls_ref[...] * out_ref[...]


def kernel(input, dw_w, dw_b, ln_w, ln_b, router_w, w1, b1, w2, b2, layer_scale):
    xh = jnp.transpose(input, (0, 2, 3, 1))
    dwk = jnp.transpose(dw_w[:, 0], (1, 2, 0))
    ls_row = layer_scale.reshape(1, _C)

    xn, gates = pl.pallas_call(
        _stage1_body,
        out_shape=[
            jax.ShapeDtypeStruct((_T, _C), jnp.float32),
            jax.ShapeDtypeStruct((_T, _E), jnp.float32),
        ],
    )(xh, dwk, dw_b, ln_w, ln_b, router_w)

    xh_flat = xh.reshape(_T, _C)
    out_flat = pl.pallas_call(
        _expert_body,
        grid=(_T // _TB, _E),
        in_specs=[
            pl.BlockSpec((_TB, _C), lambda t, e: (t, 0)),
            pl.BlockSpec((1, _C, _HID), lambda t, e: (e, 0, 0)),
            pl.BlockSpec((1, _HID), lambda t, e: (e, 0)),
            pl.BlockSpec((1, _HID, _C), lambda t, e: (e, 0, 0)),
            pl.BlockSpec((1, _C), lambda t, e: (e, 0)),
            pl.BlockSpec((_TB, _E), lambda t, e: (t, 0)),
            pl.BlockSpec((_TB, _C), lambda t, e: (t, 0)),
            pl.BlockSpec((1, _C), lambda t, e: (0, 0)),
        ],
        out_specs=pl.BlockSpec((_TB, _C), lambda t, e: (t, 0)),
        out_shape=jax.ShapeDtypeStruct((_T, _C), jnp.float32),
    )(xn, w1, b1, w2, b2, gates, xh_flat, ls_row)

    out = out_flat.reshape(_B, _H, _W, _C)
    return jnp.transpose(out, (0, 3, 1, 2))


# speedup vs baseline: 3.0019x; 3.0019x over previous
import jax
import jax.numpy as jnp
from jax import lax
from jax.experimental import pallas as pl
from jax.experimental.pallas import tpu as pltpu

_B, _C, _H, _W = 8, 384, 14, 14
_E, _K, _R = 8, 2, 4
_HW = _H * _W
_T = _B * _HW
_HID = _R * _C
_CAP = int(1.25 * _T * _K / _E)


def _stage1_body(x3_ref, dwk_ref, dwb_ref, lnw_ref, lnb_ref, rw_ref,
                 xn_ref, gates_ref):
    x = x3_ref[...]
    rpos = lax.broadcasted_iota(jnp.int32, (_HW, 1), 0)
    wpos = rpos % _W
    hpos = rpos // _W
    acc = jnp.zeros((_B, _HW, _C), jnp.float32)
    for di in range(-3, 4):
        for dj in range(-3, 4):
            s = di * _W + dj
            tap = (di + 3) * 7 + (dj + 3)
            shifted = pltpu.roll(x, (-s) % _HW, axis=1)
            valid = ((wpos + dj >= 0) & (wpos + dj < _W)
                     & (hpos + di >= 0) & (hpos + di < _H))
            mk = jnp.where(valid, 1.0, 0.0) * dwk_ref[tap]
            acc = acc + shifted * mk
    y = acc + dwb_ref[...]

    mu = jnp.mean(y, axis=2, keepdims=True)
    var = jnp.mean((y - mu) * (y - mu), axis=2, keepdims=True)
    xn3 = (y - mu) * lax.rsqrt(var + 1e-6) * lnw_ref[...] + lnb_ref[...]
    xn_ref[...] = xn3

    e_iota = lax.broadcasted_iota(jnp.int32, (_HW, _E), 1)
    p1s, i1s, i2s, w1s, w2s, masks = [], [], [], [], [], []
    for b in range(_B):
        logits = jnp.dot(xn3[b], rw_ref[...], preferred_element_type=jnp.float32)
        m = jnp.max(logits, axis=1, keepdims=True)
        ex = jnp.exp(logits - m)
        probs = ex / jnp.sum(ex, axis=1, keepdims=True)
        p1 = jnp.max(probs, axis=1, keepdims=True)
        i1 = jnp.min(jnp.where(probs == p1, e_iota, _E), axis=1, keepdims=True)
        probs2 = jnp.where(e_iota == i1, -1.0, probs)
        p2 = jnp.max(probs2, axis=1, keepdims=True)
        i2 = jnp.min(jnp.where(probs2 == p2, e_iota, _E), axis=1, keepdims=True)
        denom = p1 + p2
        p1s.append(p1); i1s.append(i1); i2s.append(i2)
        w1s.append(p1 / denom); w2s.append(p2 / denom)
        masks.append(((e_iota == i1) | (e_iota == i2)).astype(jnp.float32))

    pT = jnp.concatenate(p1s, axis=1)
    p2d = jnp.transpose(pT)
    b_iota_col = lax.broadcasted_iota(jnp.int32, (_HW, _B), 1)
    b_iota_row = lax.broadcasted_iota(jnp.int32, (_B, _HW), 0)
    ri = lax.broadcasted_iota(jnp.int32, (_HW, _HW), 0)
    ci = lax.broadcasted_iota(jnp.int32, (_HW, _HW), 1)

    for br in range(_B):
        prow = jnp.sum(jnp.where(b_iota_col == br, pT, 0.0), axis=1,
                       keepdims=True)
        rank = jnp.zeros((_HW, _E), jnp.float32)
        for bc in range(_B):
            pcol = jnp.sum(jnp.where(b_iota_row == bc, p2d, 0.0), axis=0,
                           keepdims=True)
            gt = pcol > prow
            eq = pcol == prow
            if bc < br:
                cmpb = gt | eq
            elif bc > br:
                cmpb = gt
            else:
                cmpb = gt | (eq & (ci < ri))
            rank = rank + jnp.dot(cmpb.astype(jnp.float32), masks[bc],
                                  preferred_element_type=jnp.float32)
        r1 = jnp.sum(rank * (e_iota == i1s[br]), axis=1, keepdims=True)
        r2 = jnp.sum(rank * (e_iota == i2s[br]), axis=1, keepdims=True)
        g1 = jnp.where(r1 < _CAP, w1s[br], 0.0)
        g2 = jnp.where(r2 < _CAP, w2s[br], 0.0)
        gates_ref[br] = g1 * (e_iota == i1s[br]) + g2 * (e_iota == i2s[br])


def _expert_body(xn_ref, w1_ref, b1_ref, w2_ref, b2_ref, gates_ref,
                 xh_ref, ls_ref, out_ref):
    e = pl.program_id(0)
    w1 = w1_ref[0]
    w2 = w2_ref[0]
    e_iota = lax.broadcasted_iota(jnp.int32, (_HW, _E), 1)
    for b in range(_B):
        xb = xn_ref[b]
        h = jnp.dot(xb, w1, preferred_element_type=jnp.float32) + b1_ref[0]
        h = 0.5 * h * (1.0 + lax.erf(h * 0.7071067811865476))
        y = jnp.dot(h, w2, preferred_element_type=jnp.float32) + b2_ref[0]
        ge = jnp.sum(jnp.where(e_iota == e, gates_ref[b], 0.0),
                     axis=1, keepdims=True)
        contrib = ge * y

        @pl.when(e == 0)
        def _():
            out_ref[b] = contrib

        @pl.when(e != 0)
        def _():
            out_ref[b] += contrib

        @pl.when(e == _E - 1)
        def _():
            out_ref[b] = xh_ref[b] + xb + ls_ref[...] * out_ref[b]


def kernel(input, dw_w, dw_b, ln_w, ln_b, router_w, w1, b1, w2, b2, layer_scale):
    x3 = jnp.transpose(input, (0, 2, 3, 1)).reshape(_B, _HW, _C)
    dwk = jnp.transpose(dw_w[:, 0], (1, 2, 0)).reshape(49, 1, _C)
    ls_row = layer_scale.reshape(1, _C)

    xn, gates = pl.pallas_call(
        _stage1_body,
        out_shape=[
            jax.ShapeDtypeStruct((_B, _HW, _C), jnp.float32),
            jax.ShapeDtypeStruct((_B, _HW, _E), jnp.float32),
        ],
    )(x3, dwk, dw_b, ln_w, ln_b, router_w)

    out3 = pl.pallas_call(
        _expert_body,
        grid=(_E,),
        in_specs=[
            pl.BlockSpec((_B, _HW, _C), lambda e: (0, 0, 0)),
            pl.BlockSpec((1, _C, _HID), lambda e: (e, 0, 0)),
            pl.BlockSpec((1, 1, _HID), lambda e: (e, 0, 0)),
            pl.BlockSpec((1, _HID, _C), lambda e: (e, 0, 0)),
            pl.BlockSpec((1, 1, _C), lambda e: (e, 0, 0)),
            pl.BlockSpec((_B, _HW, _E), lambda e: (0, 0, 0)),
            pl.BlockSpec((_B, _HW, _C), lambda e: (0, 0, 0)),
            pl.BlockSpec((1, _C), lambda e: (0, 0)),
        ],
        out_specs=pl.BlockSpec((_B, _HW, _C), lambda e: (0, 0, 0)),
        out_shape=jax.ShapeDtypeStruct((_B, _HW, _C), jnp.float32),
    )(xn, w1, b1.reshape(_E, 1, _HID), w2, b2.reshape(_E, 1, _C),
      gates, x3, ls_row)

    out = out3.reshape(_B, _H, _W, _C)
    return jnp.transpose(out, (0, 3, 1, 2))
